# baseline (speedup 1.0000x reference)
import functools

import jax
import jax.numpy as jnp
from jax.experimental import pallas as pl
from jax.experimental.pallas import tpu as pltpu

_B, _S, _D = 1, 2048, 1024
_H = 16
_HD = _D // _H
_CHUNK = 256
_NC = _S // _CHUNK
_TOPK = 5
_SCALE = _HD ** -0.5
_HP = _H // 2


def _qkv_kernel(x_ref, wq_ref, wk_ref, wv_ref, q_ref, k_ref, v_ref):
    x = x_ref[...]
    dn = (((1,), (1,)), ((), ()))
    q_ref[...] = jax.lax.dot_general(x, wq_ref[...], dn,
                                     preferred_element_type=jnp.float32) * _SCALE
    k_ref[...] = jax.lax.dot_general(x, wk_ref[...], dn,
                                     preferred_element_type=jnp.float32)
    v_ref[...] = jax.lax.dot_general(x, wv_ref[...], dn,
                                     preferred_element_type=jnp.float32)


def _select(sims):
    col = jax.lax.broadcasted_iota(jnp.int32, (_CHUNK, _NC), 1)
    cols = []
    for c in range(_NC):
        sc = sims[:, c:c + 1]
        beats = (sims > sc) | ((sims == sc) & (col < c))
        rank = jnp.sum(beats.astype(jnp.int32), axis=1, keepdims=True)
        cols.append((rank < _TOPK).astype(jnp.float32))
    return jnp.concatenate(cols, axis=1)


def _attn_kernel(q_ref, k_ref, vx_ref, ck_ref, o_ref):
    qc = pl.program_id(1)
    q = q_ref[...]
    ck = ck_ref[0]

    dn_t = (((1,), (1,)), ((), ()))
    dn_n = (((1,), (0,)), ((), ()))

    col8 = jax.lax.broadcasted_iota(jnp.int32, (_CHUNK, _NC), 1)
    ri = jax.lax.broadcasted_iota(jnp.int32, (_CHUNK, _CHUNK), 0)
    ci = jax.lax.broadcasted_iota(jnp.int32, (_CHUNK, _CHUNK), 1)
    tri = (ci <= ri).astype(jnp.float32)

    outs = []
    for h2 in range(2):
        qh = q[:, h2 * _HD:(h2 + 1) * _HD]
        ckh = ck[:, h2 * _HD:(h2 + 1) * _HD]
        sims = jax.lax.dot_general(qh, ckh, dn_t,
                                   preferred_element_type=jnp.float32)
        sel = _select(sims)

        def body(kc, acc):
            kh = k_ref[pl.ds(kc * _CHUNK, _CHUNK),
                       h2 * _HD:(h2 + 1) * _HD]
            vh = vx_ref[pl.ds(kc * _CHUNK, _CHUNK),
                        h2 * 2 * _HD:(h2 + 1) * 2 * _HD]
            s = jax.lax.dot_general(qh, kh, dn_t,
                                    preferred_element_type=jnp.float32)
            p = jnp.exp(s)
            allowed = jnp.sum(sel * (col8 == kc), axis=1, keepdims=True)
            p = p * allowed
            return acc + jax.lax.dot_general(
                p, vh, dn_n, preferred_element_type=jnp.float32)

        acc = jnp.zeros((_CHUNK, 2 * _HD), jnp.float32)
        acc = jax.lax.fori_loop(0, qc, body, acc)

        kh = k_ref[pl.ds(qc * _CHUNK, _CHUNK), h2 * _HD:(h2 + 1) * _HD]
        vh = vx_ref[pl.ds(qc * _CHUNK, _CHUNK),
                    h2 * 2 * _HD:(h2 + 1) * 2 * _HD]
        s = jax.lax.dot_general(qh, kh, dn_t,
                                preferred_element_type=jnp.float32)
        p = jnp.exp(s) * tri
        allowed = jnp.sum(sel * (col8 == qc), axis=1, keepdims=True)
        p = p * allowed
        acc = acc + jax.lax.dot_general(p, vh, dn_n,
                                        preferred_element_type=jnp.float32)

        pv = acc[:, :_HD]
        l = acc[:, _HD:_HD + 1]

        ones8 = jnp.ones((8, _S), jnp.float32)
        sv = jax.lax.dot_general(ones8, vx_ref[:, h2 * 2 * _HD:
                                               (h2 + 1) * 2 * _HD],
                                 dn_n, preferred_element_type=jnp.float32)
        vmean = sv[0:1, :_HD] * (1.0 / _S)
        deg = (l == 0.0).astype(jnp.float32)
        safe_l = l + deg
        outs.append(pv / safe_l * (1.0 - deg) + vmean * deg)

    o_ref[...] = jnp.concatenate(outs, axis=1)


def _oproj_kernel(a_ref, wo_ref, o_ref):
    o_ref[...] = jax.lax.dot_general(
        a_ref[...], wo_ref[...], (((1,), (1,)), ((), ())),
        preferred_element_type=jnp.float32)


def kernel(x, Wq, Wk, Wv, Wo):
    x2 = x.reshape(_S, _D)
    f32 = jnp.float32

    q, k, v = pl.pallas_call(
        _qkv_kernel,
        grid=(_NC,),
        in_specs=[
            pl.BlockSpec((_CHUNK, _D), lambda i: (i, 0)),
            pl.BlockSpec((_D, _D), lambda i: (0, 0)),
            pl.BlockSpec((_D, _D), lambda i: (0, 0)),
            pl.BlockSpec((_D, _D), lambda i: (0, 0)),
        ],
        out_specs=[
            pl.BlockSpec((_CHUNK, _D), lambda i: (i, 0)),
            pl.BlockSpec((_CHUNK, _D), lambda i: (i, 0)),
            pl.BlockSpec((_CHUNK, _D), lambda i: (i, 0)),
        ],
        out_shape=[jax.ShapeDtypeStruct((_S, _D), f32)] * 3,
    )(x2, Wq, Wk, Wv)

    K4 = k.reshape(_B, _S, _H, _HD).transpose(0, 2, 1, 3)
    ck = K4.reshape(_B, _H, _NC, _CHUNK, _HD).mean(axis=3)[0]
    ckp = ck.reshape(_HP, 2, _NC, _HD).transpose(0, 2, 1, 3).reshape(
        _HP, _NC, 2 * _HD)

    v4 = v.reshape(_S, _H, _HD)
    vx = jnp.concatenate(
        [v4, jnp.ones((_S, _H, _HD), f32)], axis=2).reshape(_S, 2 * _D)

    attn = pl.pallas_call(
        _attn_kernel,
        grid=(_HP, _NC),
        in_specs=[
            pl.BlockSpec((_CHUNK, 2 * _HD), lambda hp, qc: (qc, hp)),
            pl.BlockSpec((_S, 2 * _HD), lambda hp, qc: (0, hp)),
            pl.BlockSpec((_S, 4 * _HD), lambda hp, qc: (0, hp)),
            pl.BlockSpec((1, _NC, 2 * _HD), lambda hp, qc: (hp, 0, 0)),
        ],
        out_specs=pl.BlockSpec((_CHUNK, 2 * _HD), lambda hp, qc: (qc, hp)),
        out_shape=jax.ShapeDtypeStruct((_S, _D), f32),
    )(q, k, vx, ckp)

    out = pl.pallas_call(
        _oproj_kernel,
        grid=(_NC,),
        in_specs=[
            pl.BlockSpec((_CHUNK, _D), lambda i: (i, 0)),
            pl.BlockSpec((_D, _D), lambda i: (0, 0)),
        ],
        out_specs=pl.BlockSpec((_CHUNK, _D), lambda i: (i, 0)),
        out_shape=jax.ShapeDtypeStruct((_S, _D), f32),
    )(attn, Wo)

    return out.reshape(_B, _S, _D)
